# baseline (speedup 1.0000x reference)
import functools
import math

import jax
import jax.numpy as jnp
from jax import lax
from jax.experimental import pallas as pl
from jax.experimental.pallas import tpu as pltpu
from jax.experimental.pallas import tpu_sc as plsc

B, H, L, D_K, ROUNDS, BL = 1, 16, 2048, 64, 2, 64
BH = B * H
CH = 2 * BL
W = 2 * CH
NB2 = L // CH
NBUCK = 2 * NB2
NEG_BIG = -1000000000.0
NEG_SELF = -100000.0
LN2 = math.log(2.0)
PD = 2 * D_K
SD = 2 * D_K
NW = 32
GCH = 128


def _iota(shape, dim):
    return lax.broadcasted_iota(jnp.int32, shape, dim).astype(jnp.float32)


def _dotT(a, b, precision=jax.lax.Precision.DEFAULT):
    return lax.dot_general(a, b, (((0,), (0,)), ((), ())),
                           precision=precision,
                           preferred_element_type=jnp.float32)


def _dot(a, b, precision=jax.lax.Precision.DEFAULT):
    return lax.dot_general(a, b, (((1,), (0,)), ((), ())),
                           precision=precision,
                           preferred_element_type=jnp.float32)


def _argmax_lanes(x):
    vmax = jnp.max(x, axis=1, keepdims=True)
    cols = x.shape[1]
    idx = jnp.where(x == vmax, _iota(x.shape, 1), float(cols))
    return jnp.min(idx, axis=1, keepdims=True)


def _counting_sort(hash_col, tril_ch, tril_nb, triu_bk):
    oh = (hash_col == _iota((L, NBUCK), 1)).astype(jnp.float32)
    ranks, totals = [], []
    for n in range(NB2):
        blk = oh[n * CH:(n + 1) * CH]
        ranks.append(_dot(tril_ch, blk))
        totals.append(jnp.sum(blk, axis=0, keepdims=True))
    bt = jnp.concatenate(totals, axis=0)
    off = _dot(tril_nb, bt)
    counts = jnp.sum(bt, axis=0, keepdims=True)
    base = _dot(counts, triu_bk, precision=jax.lax.Precision.HIGHEST)
    pos = []
    for n in range(NB2):
        blk = oh[n * CH:(n + 1) * CH]
        p = jnp.sum(blk * (base + off[n:n + 1, :] + ranks[n]),
                    axis=1, keepdims=True)
        pos.append(p)
    return jnp.concatenate(pos, axis=0)


def _split2(x):
    hi = x.astype(jnp.bfloat16).astype(jnp.float32)
    return hi, x - hi


def _prep_body(q_ref, v_ref, rm_ref, pay_ref, gcol_ref, scol_ref, sme_ref):
    bh = pl.program_id(0)
    q = q_ref[...]
    v = v_ref[...]
    rm = rm_ref[...]

    qn = q / jnp.sqrt(jnp.sum(q * q, axis=1, keepdims=True))
    rmn = rm / jnp.sqrt(jnp.sum(rm * rm, axis=0, keepdims=True))
    h = _dot(qn, rmn)

    tril_ch = (_iota((CH, CH), 0) > _iota((CH, CH), 1)).astype(jnp.float32)
    tril_nb = (_iota((NB2, NB2), 0) > _iota((NB2, NB2), 1)).astype(jnp.float32)
    triu_bk = (_iota((NBUCK, NBUCK), 0) < _iota((NBUCK, NBUCK), 1)).astype(jnp.float32)
    iota8 = jnp.concatenate(
        [_iota((L, 1), 0), jnp.zeros((L, 7), jnp.float32)], axis=1)

    hashes, poss, chunks = [], [], []
    for r in range(ROUNDS):
        hr = h[:, r * NB2:(r + 1) * NB2]
        hcat = jnp.concatenate([hr, -hr], axis=1)
        hsh = _argmax_lanes(hcat)
        pos = _counting_sort(hsh, tril_ch, tril_nb, triu_bk)
        hashes.append(hsh)
        poss.append(pos)
        chunks.append(jnp.floor(pos * (1.0 / CH)))

    meta = jnp.concatenate(
        [iota8[:, 0:1], hashes[0], hashes[1], chunks[0], chunks[1],
         jnp.zeros((L, 3), jnp.float32)], axis=1)
    pay_ref[...] = jnp.concatenate([qn, v], axis=1)

    bhf = bh.astype(jnp.float32)
    mh, mlo = _split2(meta)
    meta16 = jnp.concatenate([mh, mlo], axis=1)
    for r in range(ROUNDS):
        scol_ref[r * L:(r + 1) * L, :] = jnp.concatenate(
            [poss[r] + (bhf * float(ROUNDS) + float(r)) * float(L),
             jnp.zeros((L, 7), jnp.float32)], axis=1).astype(jnp.int32)
    for r in range(ROUNDS):
        pos = poss[r]
        lo = pos - chunks[r] * float(CH)
        Elo = (lo == _iota((L, CH), 1)).astype(jnp.float32)
        for n in range(NB2):
            cmask = (chunks[r] == float(n)).astype(jnp.float32)
            sm2 = _dotT(Elo, meta16 * cmask)
            sm = sm2[:, :8] + sm2[:, 8:]
            rows = slice(r * L + n * CH, r * L + (n + 1) * CH)
            gcol_ref[rows, :] = jnp.concatenate(
                [sm[:, 0:1] + bhf * float(L),
                 jnp.zeros((CH, 7), jnp.float32)], axis=1).astype(jnp.int32)
            sme_ref[rows, :] = jnp.concatenate(
                [sm[:, 1 + r:2 + r], sm[:, 4 - r:5 - r],
                 jnp.zeros((CH, 6), jnp.float32)], axis=1)


def _attn_body(sp_ref, sme_ref, o_ref):
    sp = sp_ref[...]
    qn = sp[:, :D_K]
    v = sp[:, D_K:2 * D_K]
    eye_ch = (_iota((CH, CH), 0) == _iota((CH, CH), 1)).astype(jnp.float32)
    smeta = sme_ref[...]
    SMT = []
    for n in range(NB2):
        SMT.append(_dotT(smeta[n * CH:(n + 1) * CH], eye_ch))

    jio = _iota((CH, W), 1)
    iio = _iota((CH, W), 0)
    later1 = (jio >= float(CH)) & ((jio - float(CH)) > iio)
    later0 = (jio < float(CH)) | later1
    selfm = (jio - float(CH)) == iio
    outs = []
    for n in range(NB2):
        p = (n - 1) % NB2
        Q = qn[n * CH:(n + 1) * CH]
        K = jnp.concatenate([qn[p * CH:(p + 1) * CH], Q], axis=0)
        V = jnp.concatenate([v[p * CH:(p + 1) * CH],
                             v[n * CH:(n + 1) * CH]], axis=0)
        S = lax.dot_general(Q, K, (((1,), (1,)), ((), ())),
                            preferred_element_type=jnp.float32)
        S = S * (1.0 / math.sqrt(D_K))
        sh = smeta[n * CH:(n + 1) * CH, 0:1]
        qo = smeta[n * CH:(n + 1) * CH, 1:2]
        wT = jnp.concatenate([SMT[p], SMT[n]], axis=1)
        kh = wT[0:1, :]
        ko = wT[1:2, :]
        S = jnp.where((sh != kh) | (later0 if n == 0 else later1),
                      NEG_BIG, S)
        S = jnp.where(selfm, NEG_SELF, S)
        qom1 = qo - 1.0
        qom1 = jnp.where(qom1 < 0.0, qom1 + float(NB2), qom1)
        dup = (ko == qo) | (ko == qom1)
        S = jnp.where(dup, S - LN2, S)
        m = jnp.max(S, axis=1, keepdims=True)
        wgt = jnp.exp(S - m)
        lsum = jnp.sum(wgt, axis=1, keepdims=True)
        acc = _dot(wgt, V)
        outs.append(jnp.concatenate(
            [acc, m, lsum, jnp.zeros((CH, SD - D_K - 2), jnp.float32)],
            axis=1))
    o_ref[...] = jnp.concatenate(outs, axis=0)


def _combine_body(st_ref, o_ref):
    x = st_ref[...]
    s0, s1 = x[:L], x[L:]
    a0, m0, l0 = s0[:, :D_K], s0[:, D_K:D_K + 1], s0[:, D_K + 1:D_K + 2]
    a1, m1, l1 = s1[:, :D_K], s1[:, D_K:D_K + 1], s1[:, D_K + 1:D_K + 2]
    mm = jnp.maximum(m0, m1)
    e0 = jnp.exp(m0 - mm)
    e1 = jnp.exp(m1 - mm)
    z = l0 * e0 + l1 * e1
    o_ref[...] = (a0 * e0 + a1 * e1) / z


def _sc_row_gather(n_out, d):
    per_w = n_out // NW
    mesh = plsc.VectorSubcoreMesh(core_axis_name="c", subcore_axis_name="s")

    @functools.partial(
        pl.kernel, mesh=mesh,
        out_type=jax.ShapeDtypeStruct((n_out, d), jnp.float32),
        scratch_types=[
            pltpu.VMEM((per_w,), jnp.int32),
            pltpu.VMEM((GCH, d), jnp.float32),
            pltpu.SemaphoreType.DMA,
        ],
    )
    def k(table_hbm, idx_hbm, out_hbm, idx_v, rows_v, sem):
        wid = lax.axis_index("s") * 2 + lax.axis_index("c")
        base = wid * per_w
        pltpu.sync_copy(idx_hbm.at[pl.ds(base, per_w)], idx_v)
        for j in range(per_w // GCH):
            pltpu.async_copy(
                table_hbm.at[idx_v.at[pl.ds(j * GCH, GCH)]], rows_v,
                sem).wait()
            pltpu.sync_copy(rows_v, out_hbm.at[pl.ds(base + j * GCH, GCH)])

    return k


def _tc_prep(q2, v2, rm2, interpret=False):
    return pl.pallas_call(
        _prep_body,
        grid=(BH,),
        in_specs=[
            pl.BlockSpec((L, D_K), lambda i: (i, 0)),
            pl.BlockSpec((L, D_K), lambda i: (i, 0)),
            pl.BlockSpec((D_K, NBUCK), lambda i: (i, 0)),
        ],
        out_specs=[
            pl.BlockSpec((L, PD), lambda i: (i, 0)),
            pl.BlockSpec((ROUNDS * L, 8), lambda i: (i, 0)),
            pl.BlockSpec((ROUNDS * L, 8), lambda i: (i, 0)),
            pl.BlockSpec((ROUNDS * L, 8), lambda i: (i, 0)),
        ],
        out_shape=[
            jax.ShapeDtypeStruct((BH * L, PD), jnp.float32),
            jax.ShapeDtypeStruct((BH * ROUNDS * L, 8), jnp.int32),
            jax.ShapeDtypeStruct((BH * ROUNDS * L, 8), jnp.int32),
            jax.ShapeDtypeStruct((BH * ROUNDS * L, 8), jnp.float32),
        ],
        interpret=interpret,
    )(q2, v2, rm2)


def _tc_attn(spay, smeta, interpret=False):
    return pl.pallas_call(
        _attn_body,
        grid=(BH * ROUNDS,),
        in_specs=[pl.BlockSpec((L, PD), lambda i: (i, 0)),
                  pl.BlockSpec((L, 8), lambda i: (i, 0))],
        out_specs=pl.BlockSpec((L, SD), lambda i: (i, 0)),
        out_shape=jax.ShapeDtypeStruct((BH * ROUNDS * L, SD), jnp.float32),
        interpret=interpret,
    )(spay, smeta)


def _tc_combine(statso, interpret=False):
    return pl.pallas_call(
        _combine_body,
        grid=(BH,),
        in_specs=[pl.BlockSpec((ROUNDS * L, SD), lambda i: (i, 0))],
        out_specs=pl.BlockSpec((L, D_K), lambda i: (i, 0)),
        out_shape=jax.ShapeDtypeStruct((BH * L, D_K), jnp.float32),
        interpret=interpret,
    )(statso)


def kernel(query, value, rand_matrix, seed):
    q2 = query.reshape(BH * L, D_K)
    v2 = value.reshape(BH * L, D_K)
    rm2 = rand_matrix.reshape(BH * D_K, ROUNDS * NB2)
    payload, gcol, scol, smeta = _tc_prep(q2, v2, rm2)
    gidx = gcol[:, 0]
    sidx = scol[:, 0]
    spay = _sc_row_gather(BH * ROUNDS * L, PD)(payload, gidx)
    stats = _tc_attn(spay, smeta)
    statso = _sc_row_gather(BH * ROUNDS * L, SD)(stats, sidx)
    out = _tc_combine(statso)
    return out.reshape(B, H, L, D_K)


# speedup vs baseline: 1.1359x; 1.1359x over previous
import functools
import math

import jax
import jax.numpy as jnp
from jax import lax
from jax.experimental import pallas as pl
from jax.experimental.pallas import tpu as pltpu
from jax.experimental.pallas import tpu_sc as plsc

B, H, L, D_K, ROUNDS, BL = 1, 16, 2048, 64, 2, 64
BH = B * H
CH = 2 * BL
W = 2 * CH
NB2 = L // CH
NBUCK = 2 * NB2
NEG_BIG = -1000000000.0
NEG_SELF = -100000.0
LN2 = math.log(2.0)
PD = 2 * D_K
SD = 2 * D_K
NW = 32
GCH = 128


def _iota(shape, dim):
    return lax.broadcasted_iota(jnp.int32, shape, dim).astype(jnp.float32)


def _dotT(a, b, precision=jax.lax.Precision.DEFAULT):
    return lax.dot_general(a, b, (((0,), (0,)), ((), ())),
                           precision=precision,
                           preferred_element_type=jnp.float32)


def _dot(a, b, precision=jax.lax.Precision.DEFAULT):
    return lax.dot_general(a, b, (((1,), (0,)), ((), ())),
                           precision=precision,
                           preferred_element_type=jnp.float32)


def _argmax_lanes(x):
    vmax = jnp.max(x, axis=1, keepdims=True)
    cols = x.shape[1]
    idx = jnp.where(x == vmax, _iota(x.shape, 1), float(cols))
    return jnp.min(idx, axis=1, keepdims=True)


def _counting_sort(hash_col, tril_ch, tril_nb, triu_bk):
    oh = (hash_col == _iota((L, NBUCK), 1)).astype(jnp.float32)
    ranks, totals = [], []
    for n in range(NB2):
        blk = oh[n * CH:(n + 1) * CH]
        ranks.append(_dot(tril_ch, blk))
        totals.append(jnp.sum(blk, axis=0, keepdims=True))
    bt = jnp.concatenate(totals, axis=0)
    off = _dot(tril_nb, bt)
    counts = jnp.sum(bt, axis=0, keepdims=True)
    base = _dot(counts, triu_bk, precision=jax.lax.Precision.HIGHEST)
    pos = []
    for n in range(NB2):
        blk = oh[n * CH:(n + 1) * CH]
        p = jnp.sum(blk * (base + off[n:n + 1, :] + ranks[n]),
                    axis=1, keepdims=True)
        pos.append(p)
    return jnp.concatenate(pos, axis=0)


def _split2(x):
    hi = x.astype(jnp.bfloat16).astype(jnp.float32)
    return hi, x - hi


def _prep_body(q_ref, v_ref, rm_ref, pay_ref, gcol_ref, scol_ref, sme_ref):
    bh = pl.program_id(0)
    q = q_ref[...]
    v = v_ref[...]
    rm = rm_ref[...]

    qn = q / jnp.sqrt(jnp.sum(q * q, axis=1, keepdims=True))
    rmn = rm / jnp.sqrt(jnp.sum(rm * rm, axis=0, keepdims=True))
    h = _dot(qn, rmn)

    tril_ch = (_iota((CH, CH), 0) > _iota((CH, CH), 1)).astype(jnp.float32)
    tril_nb = (_iota((NB2, NB2), 0) > _iota((NB2, NB2), 1)).astype(jnp.float32)
    triu_bk = (_iota((NBUCK, NBUCK), 0) < _iota((NBUCK, NBUCK), 1)).astype(jnp.float32)
    iota8 = jnp.concatenate(
        [_iota((L, 1), 0), jnp.zeros((L, 7), jnp.float32)], axis=1)

    hashes, poss, chunks = [], [], []
    for r in range(ROUNDS):
        hr = h[:, r * NB2:(r + 1) * NB2]
        hcat = jnp.concatenate([hr, -hr], axis=1)
        hsh = _argmax_lanes(hcat)
        pos = _counting_sort(hsh, tril_ch, tril_nb, triu_bk)
        hashes.append(hsh)
        poss.append(pos)
        chunks.append(jnp.floor(pos * (1.0 / CH)))

    meta = jnp.concatenate(
        [iota8[:, 0:1], hashes[0], hashes[1], chunks[0], chunks[1],
         jnp.zeros((L, 3), jnp.float32)], axis=1)
    pay_ref[...] = jnp.concatenate([qn, v], axis=1)

    bhf = bh.astype(jnp.float32)
    mh, mlo = _split2(meta)
    meta16 = jnp.concatenate([mh, mlo], axis=1)
    for r in range(ROUNDS):
        scol_ref[r * L:(r + 1) * L, :] = jnp.concatenate(
            [poss[r] + (bhf * float(ROUNDS) + float(r)) * float(L),
             jnp.zeros((L, 7), jnp.float32)], axis=1).astype(jnp.int32)
    for r in range(ROUNDS):
        pos = poss[r]
        for n in range(NB2):
            E = (pos == (n * CH + _iota((L, CH), 1))).astype(jnp.float32)
            sm2 = _dotT(E, meta16)
            sm = sm2[:, :8] + sm2[:, 8:]
            rows = slice(r * L + n * CH, r * L + (n + 1) * CH)
            gcol_ref[rows, :] = jnp.concatenate(
                [sm[:, 0:1] + bhf * float(L),
                 jnp.zeros((CH, 7), jnp.float32)], axis=1).astype(jnp.int32)
            sme_ref[rows, :] = jnp.concatenate(
                [sm[:, 1 + r:2 + r], sm[:, 4 - r:5 - r],
                 jnp.zeros((CH, 6), jnp.float32)], axis=1)


def _attn_body(sp_ref, sme_ref, o_ref):
    sp = sp_ref[...]
    qn = sp[:, :D_K]
    v = sp[:, D_K:2 * D_K]
    eye_ch = (_iota((CH, CH), 0) == _iota((CH, CH), 1)).astype(jnp.float32)
    smeta = sme_ref[...]
    SMT = []
    for n in range(NB2):
        SMT.append(_dotT(smeta[n * CH:(n + 1) * CH], eye_ch))

    jio = _iota((CH, W), 1)
    iio = _iota((CH, W), 0)
    later1 = (jio >= float(CH)) & ((jio - float(CH)) > iio)
    later0 = (jio < float(CH)) | later1
    selfm = (jio - float(CH)) == iio
    outs = []
    for n in range(NB2):
        p = (n - 1) % NB2
        Q = qn[n * CH:(n + 1) * CH]
        K = jnp.concatenate([qn[p * CH:(p + 1) * CH], Q], axis=0)
        V = jnp.concatenate([v[p * CH:(p + 1) * CH],
                             v[n * CH:(n + 1) * CH]], axis=0)
        S = lax.dot_general(Q, K, (((1,), (1,)), ((), ())),
                            preferred_element_type=jnp.float32)
        S = S * (1.0 / math.sqrt(D_K))
        sh = smeta[n * CH:(n + 1) * CH, 0:1]
        qo = smeta[n * CH:(n + 1) * CH, 1:2]
        wT = jnp.concatenate([SMT[p], SMT[n]], axis=1)
        kh = wT[0:1, :]
        ko = wT[1:2, :]
        S = jnp.where((sh != kh) | (later0 if n == 0 else later1),
                      NEG_BIG, S)
        S = jnp.where(selfm, NEG_SELF, S)
        qom1 = qo - 1.0
        qom1 = jnp.where(qom1 < 0.0, qom1 + float(NB2), qom1)
        dup = (ko == qo) | (ko == qom1)
        S = jnp.where(dup, S - LN2, S)
        m = jnp.max(S, axis=1, keepdims=True)
        wgt = jnp.exp(S - m)
        lsum = jnp.sum(wgt, axis=1, keepdims=True)
        acc = _dot(wgt, V)
        outs.append(jnp.concatenate(
            [acc, m, lsum, jnp.zeros((CH, SD - D_K - 2), jnp.float32)],
            axis=1))
    o_ref[...] = jnp.concatenate(outs, axis=0)


def _combine_body(st_ref, o_ref):
    x = st_ref[...]
    s0, s1 = x[:L], x[L:]
    a0, m0, l0 = s0[:, :D_K], s0[:, D_K:D_K + 1], s0[:, D_K + 1:D_K + 2]
    a1, m1, l1 = s1[:, :D_K], s1[:, D_K:D_K + 1], s1[:, D_K + 1:D_K + 2]
    mm = jnp.maximum(m0, m1)
    e0 = jnp.exp(m0 - mm)
    e1 = jnp.exp(m1 - mm)
    z = l0 * e0 + l1 * e1
    o_ref[...] = (a0 * e0 + a1 * e1) / z


def _sc_row_gather(n_out, d):
    per_w = n_out // NW
    mesh = plsc.VectorSubcoreMesh(core_axis_name="c", subcore_axis_name="s")

    @functools.partial(
        pl.kernel, mesh=mesh,
        out_type=jax.ShapeDtypeStruct((n_out, d), jnp.float32),
        scratch_types=[
            pltpu.VMEM((per_w,), jnp.int32),
            pltpu.VMEM((GCH, d), jnp.float32),
            pltpu.SemaphoreType.DMA,
        ],
    )
    def k(table_hbm, idx_hbm, out_hbm, idx_v, rows_v, sem):
        wid = lax.axis_index("s") * 2 + lax.axis_index("c")
        base = wid * per_w
        pltpu.sync_copy(idx_hbm.at[pl.ds(base, per_w)], idx_v)
        for j in range(per_w // GCH):
            pltpu.async_copy(
                table_hbm.at[idx_v.at[pl.ds(j * GCH, GCH)]], rows_v,
                sem).wait()
            pltpu.sync_copy(rows_v, out_hbm.at[pl.ds(base + j * GCH, GCH)])

    return k


def _tc_prep(q2, v2, rm2, interpret=False):
    return pl.pallas_call(
        _prep_body,
        grid=(BH,),
        in_specs=[
            pl.BlockSpec((L, D_K), lambda i: (i, 0)),
            pl.BlockSpec((L, D_K), lambda i: (i, 0)),
            pl.BlockSpec((D_K, NBUCK), lambda i: (i, 0)),
        ],
        out_specs=[
            pl.BlockSpec((L, PD), lambda i: (i, 0)),
            pl.BlockSpec((ROUNDS * L, 8), lambda i: (i, 0)),
            pl.BlockSpec((ROUNDS * L, 8), lambda i: (i, 0)),
            pl.BlockSpec((ROUNDS * L, 8), lambda i: (i, 0)),
        ],
        out_shape=[
            jax.ShapeDtypeStruct((BH * L, PD), jnp.float32),
            jax.ShapeDtypeStruct((BH * ROUNDS * L, 8), jnp.int32),
            jax.ShapeDtypeStruct((BH * ROUNDS * L, 8), jnp.int32),
            jax.ShapeDtypeStruct((BH * ROUNDS * L, 8), jnp.float32),
        ],
        interpret=interpret,
    )(q2, v2, rm2)


def _tc_attn(spay, smeta, interpret=False):
    return pl.pallas_call(
        _attn_body,
        grid=(BH * ROUNDS,),
        in_specs=[pl.BlockSpec((L, PD), lambda i: (i, 0)),
                  pl.BlockSpec((L, 8), lambda i: (i, 0))],
        out_specs=pl.BlockSpec((L, SD), lambda i: (i, 0)),
        out_shape=jax.ShapeDtypeStruct((BH * ROUNDS * L, SD), jnp.float32),
        interpret=interpret,
    )(spay, smeta)


def _tc_combine(statso, interpret=False):
    return pl.pallas_call(
        _combine_body,
        grid=(BH,),
        in_specs=[pl.BlockSpec((ROUNDS * L, SD), lambda i: (i, 0))],
        out_specs=pl.BlockSpec((L, D_K), lambda i: (i, 0)),
        out_shape=jax.ShapeDtypeStruct((BH * L, D_K), jnp.float32),
        interpret=interpret,
    )(statso)


def kernel(query, value, rand_matrix, seed):
    q2 = query.reshape(BH * L, D_K)
    v2 = value.reshape(BH * L, D_K)
    rm2 = rand_matrix.reshape(BH * D_K, ROUNDS * NB2)
    payload, gcol, scol, smeta = _tc_prep(q2, v2, rm2)
    gidx = gcol[:, 0]
    sidx = scol[:, 0]
    spay = _sc_row_gather(BH * ROUNDS * L, PD)(payload, gidx)
    stats = _tc_attn(spay, smeta)
    statso = _sc_row_gather(BH * ROUNDS * L, SD)(stats, sidx)
    out = _tc_combine(statso)
    return out.reshape(B, H, L, D_K)


# speedup vs baseline: 1.1769x; 1.0360x over previous
import functools
import math

import jax
import jax.numpy as jnp
from jax import lax
from jax.experimental import pallas as pl
from jax.experimental.pallas import tpu as pltpu
from jax.experimental.pallas import tpu_sc as plsc

B, H, L, D_K, ROUNDS, BL = 1, 16, 2048, 64, 2, 64
BH = B * H
CH = 2 * BL
W = 2 * CH
NB2 = L // CH
NBUCK = 2 * NB2
NEG_BIG = -1000000000.0
NEG_SELF = -100000.0
LN2 = math.log(2.0)
PD = 2 * D_K
SD = 2 * D_K
NW = 32
GCH = 128


def _iota(shape, dim):
    return lax.broadcasted_iota(jnp.int32, shape, dim).astype(jnp.float32)


def _dotT(a, b, precision=jax.lax.Precision.DEFAULT):
    return lax.dot_general(a, b, (((0,), (0,)), ((), ())),
                           precision=precision,
                           preferred_element_type=jnp.float32)


def _dot(a, b, precision=jax.lax.Precision.DEFAULT):
    return lax.dot_general(a, b, (((1,), (0,)), ((), ())),
                           precision=precision,
                           preferred_element_type=jnp.float32)


def _argmax_lanes(x):
    vmax = jnp.max(x, axis=1, keepdims=True)
    cols = x.shape[1]
    idx = jnp.where(x == vmax, _iota(x.shape, 1), float(cols))
    return jnp.min(idx, axis=1, keepdims=True)


def _counting_sort(hash_col, tril_ch, tril_nb, triu_bk):
    oh = (hash_col == _iota((L, NBUCK), 1)).astype(jnp.float32)
    ranks, totals = [], []
    for n in range(NB2):
        blk = oh[n * CH:(n + 1) * CH]
        ranks.append(_dot(tril_ch, blk))
        totals.append(jnp.sum(blk, axis=0, keepdims=True))
    bt = jnp.concatenate(totals, axis=0)
    off = _dot(tril_nb, bt)
    counts = jnp.sum(bt, axis=0, keepdims=True)
    base = _dot(counts, triu_bk, precision=jax.lax.Precision.HIGHEST)
    pos = []
    for n in range(NB2):
        blk = oh[n * CH:(n + 1) * CH]
        p = jnp.sum(blk * (base + off[n:n + 1, :] + ranks[n]),
                    axis=1, keepdims=True)
        pos.append(p)
    return jnp.concatenate(pos, axis=0)


def _split2(x):
    hi = x.astype(jnp.bfloat16).astype(jnp.float32)
    return hi, x - hi


def _prep_body(q_ref, v_ref, rm_ref, pay_ref, gcol_ref, scol_ref, sme_ref):
    bh = pl.program_id(0)
    q = q_ref[...]
    v = v_ref[...]
    rm = rm_ref[...]

    qn = q / jnp.sqrt(jnp.sum(q * q, axis=1, keepdims=True))
    rmn = rm / jnp.sqrt(jnp.sum(rm * rm, axis=0, keepdims=True))
    h = _dot(qn, rmn)

    tril_ch = (_iota((CH, CH), 0) > _iota((CH, CH), 1)).astype(jnp.float32)
    tril_nb = (_iota((NB2, NB2), 0) > _iota((NB2, NB2), 1)).astype(jnp.float32)
    triu_bk = (_iota((NBUCK, NBUCK), 0) < _iota((NBUCK, NBUCK), 1)).astype(jnp.float32)
    iota8 = jnp.concatenate(
        [_iota((L, 1), 0), jnp.zeros((L, 7), jnp.float32)], axis=1)

    hashes, poss, chunks = [], [], []
    for r in range(ROUNDS):
        hr = h[:, r * NB2:(r + 1) * NB2]
        hcat = jnp.concatenate([hr, -hr], axis=1)
        hsh = _argmax_lanes(hcat)
        pos = _counting_sort(hsh, tril_ch, tril_nb, triu_bk)
        hashes.append(hsh)
        poss.append(pos)
        chunks.append(jnp.floor(pos * (1.0 / CH)))

    meta = jnp.concatenate(
        [iota8[:, 0:1], hashes[0], hashes[1], chunks[0], chunks[1],
         jnp.zeros((L, 3), jnp.float32)], axis=1)
    pay_ref[...] = jnp.concatenate([qn, v], axis=1)

    bhf = bh.astype(jnp.float32)
    mh, mlo = _split2(meta)
    meta16 = jnp.concatenate([mh, mlo], axis=1)
    for r in range(ROUNDS):
        scol_ref[r * L:(r + 1) * L, :] = jnp.concatenate(
            [poss[r] + (bhf * float(ROUNDS) + float(r)) * float(L),
             jnp.zeros((L, 7), jnp.float32)], axis=1).astype(jnp.int32)
    for r in range(ROUNDS):
        pos = poss[r]
        for n in range(NB2):
            E = (pos == (n * CH + _iota((L, CH), 1))).astype(jnp.float32)
            sm2 = _dotT(E, meta16)
            sm = sm2[:, :8] + sm2[:, 8:]
            rows = slice(r * L + n * CH, r * L + (n + 1) * CH)
            gcol_ref[rows, :] = jnp.concatenate(
                [sm[:, 0:1] + bhf * float(L),
                 jnp.zeros((CH, 7), jnp.float32)], axis=1).astype(jnp.int32)
            sme_ref[rows, :] = jnp.concatenate(
                [sm[:, 1 + r:2 + r], sm[:, 4 - r:5 - r],
                 jnp.zeros((CH, 6), jnp.float32)], axis=1)


def _attn_body(sp_ref, sme_ref, o_ref):
    sp = sp_ref[...]
    qn = sp[:, :D_K]
    v = sp[:, D_K:2 * D_K]
    eye_ch = (_iota((CH, CH), 0) == _iota((CH, CH), 1)).astype(jnp.float32)
    smeta = sme_ref[...]
    SMT = []
    for n in range(NB2):
        SMT.append(_dotT(smeta[n * CH:(n + 1) * CH], eye_ch,
                         precision=jax.lax.Precision.HIGHEST))

    jio = _iota((CH, W), 1)
    iio = _iota((CH, W), 0)
    later1 = (jio >= float(CH)) & ((jio - float(CH)) > iio)
    later0 = (jio < float(CH)) | later1
    selfm = (jio - float(CH)) == iio
    outs = []
    for n in range(NB2):
        p = (n - 1) % NB2
        Q = qn[n * CH:(n + 1) * CH]
        K = jnp.concatenate([qn[p * CH:(p + 1) * CH], Q], axis=0)
        V = jnp.concatenate([v[p * CH:(p + 1) * CH],
                             v[n * CH:(n + 1) * CH]], axis=0)
        S = lax.dot_general(Q, K, (((1,), (1,)), ((), ())),
                            preferred_element_type=jnp.float32)
        S = S * (1.0 / math.sqrt(D_K))
        sh = smeta[n * CH:(n + 1) * CH, 0:1]
        qo = smeta[n * CH:(n + 1) * CH, 1:2]
        wT = jnp.concatenate([SMT[p], SMT[n]], axis=1)
        kh = wT[0:1, :]
        ko = wT[1:2, :]
        S = jnp.where(sh != kh, NEG_BIG, S)
        S = jnp.where(later0 if n == 0 else later1, NEG_BIG, S)
        S = jnp.where(selfm, NEG_SELF, S)
        qom1 = qo - 1.0
        qom1 = jnp.where(qom1 < 0.0, qom1 + float(NB2), qom1)
        dup = (ko == qo) | (ko == qom1)
        S = jnp.where(dup, S - LN2, S)
        m = jnp.max(S, axis=1, keepdims=True)
        wgt = jnp.exp(S - m)
        lsum = jnp.sum(wgt, axis=1, keepdims=True)
        acc = _dot(wgt, V)
        outs.append(jnp.concatenate(
            [acc, m, lsum, jnp.zeros((CH, SD - D_K - 2), jnp.float32)],
            axis=1))
    o_ref[...] = jnp.concatenate(outs, axis=0)


def _combine_body(st_ref, o_ref):
    x = st_ref[...]
    s0, s1 = x[:L], x[L:]
    a0, m0, l0 = s0[:, :D_K], s0[:, D_K:D_K + 1], s0[:, D_K + 1:D_K + 2]
    a1, m1, l1 = s1[:, :D_K], s1[:, D_K:D_K + 1], s1[:, D_K + 1:D_K + 2]
    mm = jnp.maximum(m0, m1)
    e0 = jnp.exp(m0 - mm)
    e1 = jnp.exp(m1 - mm)
    z = l0 * e0 + l1 * e1
    o_ref[...] = (a0 * e0 + a1 * e1) / z


def _sc_row_gather(n_out, d):
    per_w = n_out // NW
    mesh = plsc.VectorSubcoreMesh(core_axis_name="c", subcore_axis_name="s")

    @functools.partial(
        pl.kernel, mesh=mesh,
        out_type=jax.ShapeDtypeStruct((n_out, d), jnp.float32),
        scratch_types=[
            pltpu.VMEM((per_w,), jnp.int32),
            pltpu.VMEM((GCH, d), jnp.float32),
            pltpu.SemaphoreType.DMA,
        ],
    )
    def k(table_hbm, idx_hbm, out_hbm, idx_v, rows_v, sem):
        wid = lax.axis_index("s") * 2 + lax.axis_index("c")
        base = wid * per_w
        pltpu.sync_copy(idx_hbm.at[pl.ds(base, per_w)], idx_v)
        for j in range(per_w // GCH):
            pltpu.async_copy(
                table_hbm.at[idx_v.at[pl.ds(j * GCH, GCH)]], rows_v,
                sem).wait()
            pltpu.sync_copy(rows_v, out_hbm.at[pl.ds(base + j * GCH, GCH)])

    return k


def _tc_prep(q2, v2, rm2, interpret=False):
    return pl.pallas_call(
        _prep_body,
        grid=(BH,),
        in_specs=[
            pl.BlockSpec((L, D_K), lambda i: (i, 0)),
            pl.BlockSpec((L, D_K), lambda i: (i, 0)),
            pl.BlockSpec((D_K, NBUCK), lambda i: (i, 0)),
        ],
        out_specs=[
            pl.BlockSpec((L, PD), lambda i: (i, 0)),
            pl.BlockSpec((ROUNDS * L, 8), lambda i: (i, 0)),
            pl.BlockSpec((ROUNDS * L, 8), lambda i: (i, 0)),
            pl.BlockSpec((ROUNDS * L, 8), lambda i: (i, 0)),
        ],
        out_shape=[
            jax.ShapeDtypeStruct((BH * L, PD), jnp.float32),
            jax.ShapeDtypeStruct((BH * ROUNDS * L, 8), jnp.int32),
            jax.ShapeDtypeStruct((BH * ROUNDS * L, 8), jnp.int32),
            jax.ShapeDtypeStruct((BH * ROUNDS * L, 8), jnp.float32),
        ],
        interpret=interpret,
    )(q2, v2, rm2)


def _tc_attn(spay, smeta, interpret=False):
    return pl.pallas_call(
        _attn_body,
        grid=(BH * ROUNDS,),
        in_specs=[pl.BlockSpec((L, PD), lambda i: (i, 0)),
                  pl.BlockSpec((L, 8), lambda i: (i, 0))],
        out_specs=pl.BlockSpec((L, SD), lambda i: (i, 0)),
        out_shape=jax.ShapeDtypeStruct((BH * ROUNDS * L, SD), jnp.float32),
        interpret=interpret,
    )(spay, smeta)


def _tc_combine(statso, interpret=False):
    return pl.pallas_call(
        _combine_body,
        grid=(BH,),
        in_specs=[pl.BlockSpec((ROUNDS * L, SD), lambda i: (i, 0))],
        out_specs=pl.BlockSpec((L, D_K), lambda i: (i, 0)),
        out_shape=jax.ShapeDtypeStruct((BH * L, D_K), jnp.float32),
        interpret=interpret,
    )(statso)


def kernel(query, value, rand_matrix, seed):
    q2 = query.reshape(BH * L, D_K)
    v2 = value.reshape(BH * L, D_K)
    rm2 = rand_matrix.reshape(BH * D_K, ROUNDS * NB2)
    payload, gcol, scol, smeta = _tc_prep(q2, v2, rm2)
    gidx = gcol[:, 0]
    sidx = scol[:, 0]
    spay = _sc_row_gather(BH * ROUNDS * L, PD)(payload, gidx)
    stats = _tc_attn(spay, smeta)
    statso = _sc_row_gather(BH * ROUNDS * L, SD)(stats, sidx)
    out = _tc_combine(statso)
    return out.reshape(B, H, L, D_K)
